# baseline (speedup 1.0000x reference)
import functools

import jax
import jax.numpy as jnp
from jax import lax
from jax.experimental import pallas as pl
from jax.experimental.pallas import tpu as pltpu
from jax.experimental.pallas import tpu_sc as plsc

_B = 16384
_C = 26
_V = 100000
_D = 32
_NC = 2
_NS = 16
_CHUNK = 4096
_NK = _B // _CHUNK
_OFF = (0, 33280, 66560)
_LEN = (33280, 33280, 33408)
_TPAD = 128
_BUF = 33408 + _TPAD
_L2 = 33440

_mesh = plsc.VectorSubcoreMesh(core_axis_name="c", subcore_axis_name="s")


@functools.partial(
    pl.kernel,
    mesh=_mesh,
    compiler_params=pltpu.CompilerParams(
        use_tc_tiling_on_sc=True, needs_layout_passes=False),
    out_type=jax.ShapeDtypeStruct((_C * _D, _B), jnp.float32),
    scratch_types=[
        pltpu.VMEM((_BUF,), jnp.float32),
        pltpu.VMEM((_BUF,), jnp.float32),
        pltpu.VMEM((_B,), jnp.int32),
        pltpu.VMEM((_B,), jnp.float32),
        pltpu.VMEM((2, _CHUNK), jnp.float32),
        pltpu.VMEM_SHARED((2, _NK, _CHUNK), jnp.int32),
        pltpu.SemaphoreType.DMA,
        pltpu.SemaphoreType.DMA,
        pltpu.SemaphoreType.DMA,
        pltpu.SemaphoreType.DMA,
    ],
)
def _encoder(idx_hbm, tab_hbm, tail_hbm, out_hbm, col_b0, col_b1, idx_v,
             res_v, obuf, idx_sh, colsem, idxsem, ssem, psem):
    d = lax.axis_index("c") * _NS + lax.axis_index("s")
    col_bufs = (col_b0, col_b1)
    is_pub = lax.axis_index("s") == 0

    def t_issue(c, t, b):
        pltpu.async_copy(
            tab_hbm.at[c * _D + d, pl.ds(_OFF[t], _LEN[t])],
            col_bufs[b].at[pl.ds(0, _LEN[t])], colsem)
        if t == 2:
            pltpu.async_copy(
                tail_hbm.at[c, d],
                col_bufs[b].at[pl.ds(_LEN[2], _TPAD)], colsem)

    def t_wait(t, b):
        pltpu.make_async_copy(
            tab_hbm.at[0, pl.ds(0, _LEN[t])],
            col_bufs[b].at[pl.ds(0, _LEN[t])], colsem).wait()
        if t == 2:
            pltpu.make_async_copy(
                tail_hbm.at[0, 0],
                col_bufs[b].at[pl.ds(_LEN[2], _TPAD)], colsem).wait()

    def idx_refill(slot, k):
        pltpu.async_copy(
            idx_sh.at[slot, k],
            idx_v.at[pl.ds(k * _CHUNK, _CHUNK)], idxsem)

    def idx_publish(c1, slot):
        for k in range(_NK):
            pltpu.async_copy(
                idx_hbm.at[c1, pl.ds(k * _CHUNK, _CHUNK)],
                idx_sh.at[slot, k], psem)

    def idx_publish_wait():
        for _k in range(_NK):
            pltpu.make_async_copy(
                idx_hbm.at[0, pl.ds(0, _CHUNK)],
                idx_sh.at[0, 0], psem).wait()

    def idx_wait():
        pltpu.make_async_copy(
            idx_sh.at[0, 0],
            idx_v.at[pl.ds(0, _CHUNK)], idxsem).wait()

    def store_wait():
        pltpu.make_async_copy(
            obuf.at[0], out_hbm.at[0, pl.ds(0, _CHUNK)], ssem).wait()

    def masked_vals(j, t, b):
        sl = pl.ds(j * 16, 16)
        iv = idx_v[sl]
        ivq = iv - _OFF[t]
        ext = _L2 if t == 2 else _LEN[t]
        m = plsc.bitcast(ivq, jnp.uint32) < jnp.uint32(ext)
        vals = plsc.load_gather(col_bufs[b], [ivq], mask=m)
        return sl, jnp.where(m, vals, jnp.float32(0.0))

    for k in range(_NK):
        pltpu.async_copy(
            idx_hbm.at[0, pl.ds(k * _CHUNK, _CHUNK)],
            idx_v.at[pl.ds(k * _CHUNK, _CHUNK)], idxsem)
    t_issue(0, 0, 0)
    t_issue(0, 1, 1)

    def pair_body(i, carry):
        for p in range(2):
            c = 2 * i + p
            a, b = p, 1 - p

            @pl.when(jnp.logical_and(is_pub, c + 1 < _C))
            def _():
                idx_publish(c + 1, b)
            for _k in range(_NK):
                idx_wait()
            t_wait(0, a)

            @plsc.parallel_loop(0, _B // 16, unroll=8)
            def _pass0(j):
                sl, vz = masked_vals(j, 0, a)
                res_v[sl] = vz

            t_issue(c, 2, a)
            t_wait(1, b)

            @plsc.parallel_loop(0, _B // 16, unroll=8)
            def _pass1(j):
                sl, vz = masked_vals(j, 1, b)
                plsc.addupdate(res_v.at[sl], vz)

            @pl.when(c + 1 < _C)
            def _():
                t_issue(c + 1, 0, b)
            t_wait(2, a)
            @pl.when(jnp.logical_and(is_pub, c + 1 < _C))
            def _():
                idx_publish_wait()
            plsc.subcore_barrier()

            f = c * _D + d
            for k in range(_NK):
                oslot = k % 2
                if k >= 2:
                    store_wait()
                else:
                    @pl.when(c >= 1)
                    def _():
                        store_wait()

                @plsc.parallel_loop(k * (_CHUNK // 16), (k + 1) * (_CHUNK // 16),
                                    unroll=8)
                def _pass2(j):
                    sl, vz = masked_vals(j, 2, a)
                    lo = j * 16 - k * _CHUNK
                    obuf[oslot, pl.ds(lo, 16)] = res_v[sl] + vz

                pltpu.async_copy(
                    obuf.at[oslot],
                    out_hbm.at[f, pl.ds(k * _CHUNK, _CHUNK)], ssem)

                @pl.when(c + 1 < _C)
                def _():
                    idx_refill(b, k)

            @pl.when(c + 1 < _C)
            def _():
                t_issue(c + 1, 1, a)
        return carry

    lax.fori_loop(0, _C // 2, pair_body, 0)
    store_wait()
    store_wait()


def kernel(tensor, tables):
    idx = tensor.T.astype(jnp.int32)
    tab = jnp.transpose(tables, (0, 2, 1))
    tail = jnp.pad(tab[:, :, _OFF[2] + _LEN[2]:], ((0, 0), (0, 0), (0, 96)))
    tab2 = tab.reshape(_C * _D, _V)
    out_t = _encoder(idx, tab2, tail)
    return out_t.T.reshape(_B, _C * _D)


# speedup vs baseline: 1.0346x; 1.0346x over previous
import functools

import jax
import jax.numpy as jnp
from jax import lax
from jax.experimental import pallas as pl
from jax.experimental.pallas import tpu as pltpu
from jax.experimental.pallas import tpu_sc as plsc

_B = 16384
_C = 26
_V = 100000
_D = 32
_NC = 2
_NS = 16
_CHUNK = 2048
_NK = _B // _CHUNK
_OFF = (0, 33280, 66560)
_LEN = (33280, 33280, 33408)
_TPAD = 128
_BUF = 33408 + _TPAD
_L2 = 33440

_mesh = plsc.VectorSubcoreMesh(core_axis_name="c", subcore_axis_name="s")


@functools.partial(
    pl.kernel,
    mesh=_mesh,
    compiler_params=pltpu.CompilerParams(
        use_tc_tiling_on_sc=True, needs_layout_passes=False),
    out_type=jax.ShapeDtypeStruct((_C * _D, _B), jnp.float32),
    scratch_types=[
        pltpu.VMEM((_BUF,), jnp.float32),
        pltpu.VMEM((_BUF,), jnp.float32),
        pltpu.VMEM((_B,), jnp.int32),
        pltpu.VMEM((_B,), jnp.float32),
        pltpu.VMEM((2, _CHUNK), jnp.float32),
        pltpu.VMEM_SHARED((2, _NK, _CHUNK), jnp.int32),
        pltpu.SemaphoreType.DMA,
        pltpu.SemaphoreType.DMA,
        pltpu.SemaphoreType.DMA,
        pltpu.SemaphoreType.DMA,
    ],
)
def _encoder(idx_hbm, tab_hbm, tail_hbm, out_hbm, col_b0, col_b1, idx_v,
             res_v, obuf, idx_sh, colsem, idxsem, ssem, psem):
    d = lax.axis_index("c") * _NS + lax.axis_index("s")
    col_bufs = (col_b0, col_b1)
    is_pub = lax.axis_index("s") == 0

    def t_issue(c, t, b):
        pltpu.async_copy(
            tab_hbm.at[c * _D + d, pl.ds(_OFF[t], _LEN[t])],
            col_bufs[b].at[pl.ds(0, _LEN[t])], colsem)
        if t == 2:
            pltpu.async_copy(
                tail_hbm.at[c, d],
                col_bufs[b].at[pl.ds(_LEN[2], _TPAD)], colsem)

    def t_wait(t, b):
        pltpu.make_async_copy(
            tab_hbm.at[0, pl.ds(0, _LEN[t])],
            col_bufs[b].at[pl.ds(0, _LEN[t])], colsem).wait()
        if t == 2:
            pltpu.make_async_copy(
                tail_hbm.at[0, 0],
                col_bufs[b].at[pl.ds(_LEN[2], _TPAD)], colsem).wait()

    def idx_refill(slot, k):
        pltpu.async_copy(
            idx_sh.at[slot, k],
            idx_v.at[pl.ds(k * _CHUNK, _CHUNK)], idxsem)

    def idx_publish(c1, slot):
        for k in range(_NK):
            pltpu.async_copy(
                idx_hbm.at[c1, pl.ds(k * _CHUNK, _CHUNK)],
                idx_sh.at[slot, k], psem)

    def idx_publish_wait():
        for _k in range(_NK):
            pltpu.make_async_copy(
                idx_hbm.at[0, pl.ds(0, _CHUNK)],
                idx_sh.at[0, 0], psem).wait()

    def idx_wait():
        pltpu.make_async_copy(
            idx_sh.at[0, 0],
            idx_v.at[pl.ds(0, _CHUNK)], idxsem).wait()

    def store_wait():
        pltpu.make_async_copy(
            obuf.at[0], out_hbm.at[0, pl.ds(0, _CHUNK)], ssem).wait()

    def masked_vals(j, t, b):
        sl = pl.ds(j * 16, 16)
        iv = idx_v[sl]
        ivq = iv - _OFF[t]
        ext = _L2 if t == 2 else _LEN[t]
        m = plsc.bitcast(ivq, jnp.uint32) < jnp.uint32(ext)
        vals = plsc.load_gather(col_bufs[b], [ivq], mask=m)
        return sl, jnp.where(m, vals, jnp.float32(0.0))

    for k in range(_NK):
        pltpu.async_copy(
            idx_hbm.at[0, pl.ds(k * _CHUNK, _CHUNK)],
            idx_v.at[pl.ds(k * _CHUNK, _CHUNK)], idxsem)
    t_issue(0, 0, 0)
    t_issue(0, 1, 1)

    def pair_body(i, carry):
        for p in range(2):
            c = 2 * i + p
            a, b = p, 1 - p

            @pl.when(jnp.logical_and(is_pub, c + 1 < _C))
            def _():
                idx_publish(c + 1, b)
            for _k in range(_NK):
                idx_wait()
            t_wait(0, a)

            @plsc.parallel_loop(0, _B // 16, unroll=8)
            def _pass0(j):
                sl, vz = masked_vals(j, 0, a)
                res_v[sl] = vz

            t_issue(c, 2, a)
            t_wait(1, b)

            @plsc.parallel_loop(0, _B // 16, unroll=8)
            def _pass1(j):
                sl, vz = masked_vals(j, 1, b)
                plsc.addupdate(res_v.at[sl], vz)

            @pl.when(c + 1 < _C)
            def _():
                t_issue(c + 1, 0, b)
            t_wait(2, a)
            @pl.when(jnp.logical_and(is_pub, c + 1 < _C))
            def _():
                idx_publish_wait()
            plsc.subcore_barrier()

            f = c * _D + d
            for k in range(_NK):
                oslot = k % 2
                if k >= 2:
                    store_wait()
                else:
                    @pl.when(c >= 1)
                    def _():
                        store_wait()

                @plsc.parallel_loop(k * (_CHUNK // 16), (k + 1) * (_CHUNK // 16),
                                    unroll=8)
                def _pass2(j):
                    sl, vz = masked_vals(j, 2, a)
                    lo = j * 16 - k * _CHUNK
                    obuf[oslot, pl.ds(lo, 16)] = res_v[sl] + vz

                pltpu.async_copy(
                    obuf.at[oslot],
                    out_hbm.at[f, pl.ds(k * _CHUNK, _CHUNK)], ssem)

                @pl.when(c + 1 < _C)
                def _():
                    idx_refill(b, k)

            @pl.when(c + 1 < _C)
            def _():
                t_issue(c + 1, 1, a)
        return carry

    lax.fori_loop(0, _C // 2, pair_body, 0)
    store_wait()
    store_wait()


def kernel(tensor, tables):
    idx = tensor.T.astype(jnp.int32)
    tab = jnp.transpose(tables, (0, 2, 1))
    tail = jnp.pad(tab[:, :, _OFF[2] + _LEN[2]:], ((0, 0), (0, 0), (0, 96)))
    tab2 = tab.reshape(_C * _D, _V)
    out_t = _encoder(idx, tab2, tail)
    return out_t.T.reshape(_B, _C * _D)
